# baseline (speedup 1.0000x reference)
import functools

import jax
import jax.numpy as jnp
from jax import lax
from jax.experimental import pallas as pl
from jax.experimental.pallas import tpu as pltpu
from jax.experimental.pallas import tpu_sc as plsc

D_MODEL = 64
GROUP = 128
SCALE = 8.0

_info = plsc.get_sparse_core_info()
_NC, _NS = _info.num_cores, _info.num_subcores
_NW = _NC * _NS


@functools.lru_cache(maxsize=None)
def _make_lookup(batch: int):
    rows_per_w = batch // _NW
    n_groups = rows_per_w // GROUP
    mesh = plsc.VectorSubcoreMesh(core_axis_name="c", subcore_axis_name="s")

    @functools.partial(
        pl.kernel,
        mesh=mesh,
        out_type=jax.ShapeDtypeStruct((batch, D_MODEL), jnp.float32),
        scratch_types=[
            pltpu.VMEM((n_groups, GROUP), jnp.int32),
            pltpu.VMEM((GROUP, D_MODEL), jnp.float32),
            pltpu.SemaphoreType.DMA,
        ],
        compiler_params=pltpu.CompilerParams(use_tc_tiling_on_sc=False),
    )
    def emb_kernel(x_hbm, table_hbm, out_hbm, idx_v, rows_v, sem):
        wid = lax.axis_index("s") * _NC + lax.axis_index("c")
        base = wid * n_groups
        pltpu.sync_copy(x_hbm.at[pl.ds(base, n_groups)], idx_v)

        def per_group(g, carry):
            pltpu.async_copy(table_hbm.at[idx_v.at[g]], rows_v, sem).wait()

            def scale_row(r, c):
                for j in range(D_MODEL // 16):
                    sl = pl.ds(j * 16, 16)
                    rows_v[r, sl] = rows_v[r, sl] * SCALE
                return c

            lax.fori_loop(0, GROUP, scale_row, 0)
            pltpu.sync_copy(rows_v, out_hbm.at[pl.ds((base + g) * GROUP, GROUP)])
            return carry

        lax.fori_loop(0, n_groups, per_group, 0)

    return emb_kernel


def kernel(x, table):
    b0, b1 = x.shape
    batch = b0 * b1
    out = _make_lookup(batch)(x.reshape(batch // GROUP, GROUP), table)
    return out.reshape(b0, b1, D_MODEL)


# speedup vs baseline: 1.2043x; 1.2043x over previous
import functools

import jax
import jax.numpy as jnp
from jax import lax
from jax.experimental import pallas as pl
from jax.experimental.pallas import tpu as pltpu
from jax.experimental.pallas import tpu_sc as plsc

D_MODEL = 64
GROUP = 128
NBUF = 8
SCALE = 8.0

_info = plsc.get_sparse_core_info()
_NC, _NS = _info.num_cores, _info.num_subcores
_NW = _NC * _NS


@functools.lru_cache(maxsize=None)
def _make_lookup(batch: int):
    rows_per_w = batch // _NW
    n_groups = rows_per_w // GROUP
    n_outer = n_groups // NBUF
    mesh = plsc.VectorSubcoreMesh(core_axis_name="c", subcore_axis_name="s")

    @functools.partial(
        pl.kernel,
        mesh=mesh,
        out_type=jax.ShapeDtypeStruct((batch, D_MODEL), jnp.float32),
        scratch_types=[
            pltpu.VMEM((n_groups, GROUP), jnp.int32),
            pltpu.VMEM((NBUF, GROUP, D_MODEL), jnp.float32),
            pltpu.SemaphoreType.DMA((NBUF,)),
        ],
        compiler_params=pltpu.CompilerParams(use_tc_tiling_on_sc=False),
    )
    def emb_kernel(x_hbm, table_hbm, out_hbm, idx_v, rows_v, gsem):
        wid = lax.axis_index("s") * _NC + lax.axis_index("c")
        base = wid * n_groups
        pltpu.sync_copy(x_hbm.at[pl.ds(base, n_groups)], idx_v)

        for b in range(NBUF):
            pltpu.async_copy(table_hbm.at[idx_v.at[b]], rows_v.at[b],
                             gsem.at[b])

        def outer(t, carry):
            for b in range(NBUF):
                g = t * NBUF + b
                pltpu.make_async_copy(table_hbm.at[idx_v.at[g]],
                                      rows_v.at[b], gsem.at[b]).wait()

                @plsc.parallel_loop(0, GROUP, unroll=8)
                def scale_row(r):
                    for j in range(D_MODEL // 16):
                        sl = pl.ds(j * 16, 16)
                        rows_v[b, r, sl] = rows_v[b, r, sl] * SCALE

                pltpu.sync_copy(rows_v.at[b],
                                out_hbm.at[pl.ds((base + g) * GROUP, GROUP)])

                @pl.when(t < n_outer - 1)
                def refill():
                    pltpu.async_copy(table_hbm.at[idx_v.at[g + NBUF]],
                                     rows_v.at[b], gsem.at[b])
            return carry

        lax.fori_loop(0, n_outer, outer, 0)

    return emb_kernel


def kernel(x, table):
    b0, b1 = x.shape
    batch = b0 * b1
    out = _make_lookup(batch)(x.reshape(batch // GROUP, GROUP), table)
    return out.reshape(b0, b1, D_MODEL)
